# baseline (speedup 1.0000x reference)
import dataclasses
import functools

import jax
import jax.numpy as jnp
from jax import lax
from jax.experimental import pallas as pl
from jax.experimental.pallas import tpu as pltpu
from jax.experimental.pallas import tpu_sc as plsc

N = 10000
E = 320000
F_IN = 128
HID = 16
NCLS = 16

NC = 2
NS = 16
L = 16
NW = NC * NS
EPW = E // NW
G = 128
NB = 8
NG = 80
T = NG // NB
FULL_ROWS = EPW // G
TAIL = EPW - FULL_ROWS * G
NPAD = 10240
RPT = NPAD // NS
TSL = N // NS
ZR = 64
HNR = NPAD // L
HC = HNR // G

NWIDE = N // 8
KWIDE = F_IN * 8
DPADW = NPAD // 8

_MESH = plsc.VectorSubcoreMesh(core_axis_name="c", subcore_axis_name="s")
_SC_PARAMS = pltpu.CompilerParams(use_tc_tiling_on_sc=False)
_SC_PARAMS_NL = dataclasses.replace(_SC_PARAMS, needs_layout_passes=False)


def _stage_dst(edge_hbm, di_v, ebase, isem):
    @pl.loop(0, FULL_ROWS)
    def _(g):
        pltpu.async_copy(edge_hbm.at[1, pl.ds(ebase + g * G, G)],
                         di_v.at[g], isem)

    pltpu.sync_copy(edge_hbm.at[1, pl.ds(ebase + FULL_ROWS * G, TAIL)],
                    di_v.at[FULL_ROWS, pl.ds(0, TAIL)])
    for k in range(TAIL // L, G // L):
        di_v[FULL_ROWS, pl.ds(k * L, L)] = jnp.full((L,), N, jnp.int32)
    for g in range(FULL_ROWS + 1, NG):
        for k in range(G // L):
            di_v[g, pl.ds(k * L, L)] = jnp.full((L,), N, jnp.int32)

    @pl.loop(0, FULL_ROWS)
    def _(g):
        pltpu.make_async_copy(edge_hbm.at[1, pl.ds(ebase + g * G, G)],
                              di_v.at[g], isem).wait()


@functools.partial(
    pl.kernel,
    mesh=_MESH,
    out_type=jax.ShapeDtypeStruct((NC, HNR, L), jnp.float32),
    scratch_types=[
        pltpu.VMEM((NG * G,), jnp.int32),
        pltpu.VMEM((HNR, L), jnp.float32),
        pltpu.VMEM((HC, G), jnp.int32),
        pltpu.VMEM_SHARED((HNR, L), jnp.float32),
        pltpu.SemaphoreType.DMA,
        pltpu.SemaphoreType.DMA,
    ],
    compiler_params=_SC_PARAMS_NL,
)
def _sc_degree(edge_hbm, id_hbm, z_hbm, out_hbm, di_v, hist_v, id_v,
               acc_sh, isem, msem):
    cid = lax.axis_index("c")
    sid = lax.axis_index("s")
    wid = sid * NC + cid
    ebase = wid * EPW

    cp_i = pltpu.async_copy(id_hbm, id_v, msem)

    @pl.loop(0, FULL_ROWS)
    def _(g):
        pltpu.async_copy(edge_hbm.at[1, pl.ds(ebase + g * G, G)],
                         di_v.at[pl.ds(g * G, G)], isem)

    pltpu.sync_copy(edge_hbm.at[1, pl.ds(ebase + FULL_ROWS * G, TAIL)],
                    di_v.at[pl.ds(FULL_ROWS * G, TAIL)])
    for k in range((FULL_ROWS * G + TAIL) // L, NG * G // L):
        di_v[pl.ds(k * L, L)] = jnp.full((L,), N, jnp.int32)

    @pl.loop(0, HNR, step=ZR)
    def _(r):
        pltpu.sync_copy(z_hbm, hist_v.at[pl.ds(r, ZR)])

    pltpu.sync_copy(z_hbm.at[pl.ds(0, HNR // NS)],
                    acc_sh.at[pl.ds(sid * (HNR // NS), HNR // NS)])

    @pl.loop(0, FULL_ROWS)
    def _(g):
        pltpu.make_async_copy(edge_hbm.at[1, pl.ds(ebase + g * G, G)],
                              di_v.at[pl.ds(g * G, G)], isem).wait()
    cp_i.wait()

    ones16 = jnp.ones((L,), jnp.float32)

    @pl.loop(0, NG * G // L)
    def _(e):
        idx = di_v[pl.ds(e * L, L)]
        r = jax.lax.shift_right_logical(idx, 4)
        c = jax.lax.bitwise_and(idx, L - 1)
        plsc.addupdate_scatter(hist_v, [r, c], ones16)

    plsc.subcore_barrier()

    for c in range(HC):
        pltpu.async_copy(hist_v.at[pl.ds(c * G, G)], acc_sh.at[id_v.at[c]],
                         msem, add=True)
    for c in range(HC):
        pltpu.make_async_copy(hist_v.at[pl.ds(c * G, G)],
                              acc_sh.at[id_v.at[c]], msem).wait()

    plsc.subcore_barrier()
    pltpu.sync_copy(acc_sh.at[pl.ds(sid * (HNR // NS), HNR // NS)],
                    out_hbm.at[cid, pl.ds(sid * (HNR // NS), HNR // NS)])


@functools.partial(
    pl.kernel,
    mesh=_MESH,
    out_type=jax.ShapeDtypeStruct((NC, NPAD, HID), jnp.float32),
    scratch_types=[
        pltpu.VMEM((NG * G,), jnp.int32),
        pltpu.VMEM((NG, G), jnp.int32),
        pltpu.VMEM((NB, G, HID), jnp.float32),
        pltpu.VMEM((ZR, HID), jnp.float32),
        pltpu.VMEM_SHARED((NPAD, HID), jnp.float32),
        pltpu.VMEM_SHARED((N, HID), jnp.float32),
        pltpu.SemaphoreType.DMA,
        pltpu.SemaphoreType.DMA,
        pltpu.SemaphoreType.DMA,
        pltpu.SemaphoreType.DMA((NB,)),
        pltpu.SemaphoreType.DMA((NB,)),
    ],
    compiler_params=_SC_PARAMS,
)
def _sc_aggregate(table_hbm, edge_hbm, out_hbm,
                  si_v, di_v, rows_v, zeros_v, acc_sh, tbl_sh,
                  isem_s, isem_d, tsem, gsem, ssem):
    cid = lax.axis_index("c")
    sid = lax.axis_index("s")
    wid = sid * NC + cid
    ebase = wid * EPW

    cp_s = pltpu.async_copy(edge_hbm.at[0, pl.ds(ebase, EPW)],
                            si_v.at[pl.ds(0, EPW)], isem_s)
    cp_t = pltpu.async_copy(table_hbm.at[pl.ds(sid * TSL, TSL)],
                            tbl_sh.at[pl.ds(sid * TSL, TSL)], tsem)

    _stage_dst(edge_hbm, di_v, ebase, isem_d)

    for k in range((NG * G - EPW) // L):
        si_v[pl.ds(EPW + k * L, L)] = jnp.zeros((L,), jnp.int32)

    @pl.loop(0, ZR)
    def _(i):
        zeros_v[i] = jnp.zeros((L,), jnp.float32)

    @pl.loop(0, RPT, step=ZR)
    def _(r):
        pltpu.sync_copy(zeros_v, acc_sh.at[pl.ds(sid * RPT + r, ZR)])

    cp_s.wait()
    cp_t.wait()
    plsc.subcore_barrier()

    for b in range(NB):
        pltpu.async_copy(tbl_sh.at[si_v.at[pl.ds(b * G, G)]],
                         rows_v.at[b], gsem.at[b])

    @pl.loop(0, T - 1)
    def _(t):
        base = t * NB
        for b in range(NB):
            pltpu.make_async_copy(tbl_sh.at[si_v.at[pl.ds((base + b) * G, G)]],
                                  rows_v.at[b], gsem.at[b]).wait()
            pltpu.async_copy(rows_v.at[b], acc_sh.at[di_v.at[base + b]],
                             ssem.at[b], add=True)
        for b in range(NB):
            pltpu.make_async_copy(rows_v.at[b], acc_sh.at[di_v.at[base + b]],
                                  ssem.at[b]).wait()
            pltpu.async_copy(tbl_sh.at[si_v.at[pl.ds((base + NB + b) * G, G)]],
                             rows_v.at[b], gsem.at[b])

    for b in range(NB):
        g = (T - 1) * NB + b
        pltpu.make_async_copy(tbl_sh.at[si_v.at[pl.ds(g * G, G)]],
                              rows_v.at[b], gsem.at[b]).wait()
        pltpu.async_copy(rows_v.at[b], acc_sh.at[di_v.at[g]],
                         ssem.at[b], add=True)
    for b in range(NB):
        g = (T - 1) * NB + b
        pltpu.make_async_copy(rows_v.at[b], acc_sh.at[di_v.at[g]],
                              ssem.at[b]).wait()

    plsc.subcore_barrier()
    pltpu.sync_copy(acc_sh.at[pl.ds(sid * RPT, RPT)],
                    out_hbm.at[cid, pl.ds(sid * RPT, RPT)])


def _tc_mm1_body(xw_ref, w1b_ref, hw_ref):
    hw_ref[...] = jnp.dot(xw_ref[...], w1b_ref[...],
                          preferred_element_type=jnp.float32)


def _tc_scale_body(hw_ref, degf_ref, rmat_ref, hsw_ref, dinw_ref):
    degs = degf_ref[0, :NWIDE, :] + degf_ref[1, :NWIDE, :] + 1.0
    dinv8 = lax.rsqrt(degs)
    dinw = jnp.dot(dinv8, rmat_ref[...], preferred_element_type=jnp.float32)
    hsw_ref[...] = hw_ref[...] * dinw
    dinw_ref[...] = dinw


def _tc2_body(aggw_ref, hsw_ref, dinw_ref, w2b_ref, b1w_ref, hs2w_ref):
    a = aggw_ref[0, :NWIDE, :] + aggw_ref[1, :NWIDE, :] + hsw_ref[...]
    h1 = jnp.maximum(dinw_ref[...] * a + b1w_ref[...], 0.0)
    hs2w_ref[...] = jnp.dot(h1, w2b_ref[...],
                            preferred_element_type=jnp.float32) * dinw_ref[...]


def _tc3_body(aggw_ref, hsw_ref, dinw_ref, b2w_ref, m_ref, outw_ref):
    a = aggw_ref[0, :NWIDE, :] + aggw_ref[1, :NWIDE, :] + hsw_ref[...]
    pre = dinw_ref[...] * a + b2w_ref[...]
    m = jnp.max(pre, axis=1, keepdims=True)
    sh = pre - m
    sw = jnp.dot(jnp.exp(sh), m_ref[...], preferred_element_type=jnp.float32)
    outw_ref[...] = sh - jnp.log(sw)


def kernel(x, edge_index, W1, b1, W2, b2):
    eye8 = jnp.eye(8, dtype=jnp.float32)
    w1b = jnp.kron(eye8, W1)
    w2b = jnp.kron(eye8, W2)
    mseg = jnp.kron(eye8, jnp.ones((NCLS, NCLS), jnp.float32))
    rmat = jnp.kron(eye8, jnp.ones((1, HID), jnp.float32))
    b1w = jnp.tile(b1, 8).reshape(1, 8 * HID)
    b2w = jnp.tile(b2, 8).reshape(1, 8 * NCLS)
    xw = x.reshape(NWIDE, KWIDE)

    idrows = jnp.arange(HNR, dtype=jnp.int32).reshape(HC, G)
    zrows = jnp.zeros((ZR, L), jnp.float32)
    deg2 = _sc_degree(edge_index, idrows, zrows)

    hw = pl.pallas_call(
        _tc_mm1_body,
        out_shape=jax.ShapeDtypeStruct((NWIDE, 8 * HID), jnp.float32),
    )(xw, w1b)

    hs1w, dinw = pl.pallas_call(
        _tc_scale_body,
        out_shape=[jax.ShapeDtypeStruct((NWIDE, 8 * HID), jnp.float32),
                   jax.ShapeDtypeStruct((NWIDE, 8 * HID), jnp.float32)],
    )(hw, deg2.reshape(NC, DPADW, 8), rmat)

    agg1 = _sc_aggregate(hs1w.reshape(N, HID), edge_index)

    hs2w = pl.pallas_call(
        _tc2_body,
        out_shape=jax.ShapeDtypeStruct((NWIDE, 8 * HID), jnp.float32),
    )(agg1.reshape(NC, DPADW, 8 * HID), hs1w, dinw, w2b, b1w)

    agg2 = _sc_aggregate(hs2w.reshape(N, HID), edge_index)

    outw = pl.pallas_call(
        _tc3_body,
        out_shape=jax.ShapeDtypeStruct((NWIDE, 8 * NCLS), jnp.float32),
    )(agg2.reshape(NC, DPADW, 8 * NCLS), hs2w, dinw, b2w, mseg)

    return outw.reshape(N, NCLS)


# speedup vs baseline: 1.2301x; 1.2301x over previous
import functools

import jax
import jax.numpy as jnp
from jax import lax
from jax.experimental import pallas as pl
from jax.experimental.pallas import tpu as pltpu
from jax.experimental.pallas import tpu_sc as plsc

N = 10000
E = 320000
F_IN = 128
HID = 16
NCLS = 16

NC = 2
NS = 16
L = 16
NW = NC * NS
EPW = E // NW
G = 128
NB = 4
NG = 80
PHASES = NG // NB
FULL_ROWS = EPW // G
TAIL = EPW - FULL_ROWS * G
NPAD = 10240
RPT = NPAD // NS
TSL = N // NS
ZR = 64

NWIDE = N // 8
KWIDE = F_IN * 8
DPADW = NPAD // 8

_MESH = plsc.VectorSubcoreMesh(core_axis_name="c", subcore_axis_name="s")
_SC_PARAMS = pltpu.CompilerParams(use_tc_tiling_on_sc=False)


def _stage_dst(edge_hbm, di_v, ebase, isem):
    @pl.loop(0, FULL_ROWS)
    def _(g):
        pltpu.async_copy(edge_hbm.at[1, pl.ds(ebase + g * G, G)],
                         di_v.at[g], isem)

    pltpu.sync_copy(edge_hbm.at[1, pl.ds(ebase + FULL_ROWS * G, TAIL)],
                    di_v.at[FULL_ROWS, pl.ds(0, TAIL)])
    for k in range(TAIL // L, G // L):
        di_v[FULL_ROWS, pl.ds(k * L, L)] = jnp.full((L,), N, jnp.int32)
    for g in range(FULL_ROWS + 1, NG):
        for k in range(G // L):
            di_v[g, pl.ds(k * L, L)] = jnp.full((L,), N, jnp.int32)

    @pl.loop(0, FULL_ROWS)
    def _(g):
        pltpu.make_async_copy(edge_hbm.at[1, pl.ds(ebase + g * G, G)],
                              di_v.at[g], isem).wait()


@functools.partial(
    pl.kernel,
    mesh=_MESH,
    out_type=jax.ShapeDtypeStruct((NC, NPAD, HID), jnp.float32),
    scratch_types=[
        pltpu.VMEM((NG, G), jnp.int32),
        pltpu.VMEM((G, HID), jnp.float32),
        pltpu.VMEM((ZR, HID), jnp.float32),
        pltpu.VMEM_SHARED((NPAD, HID), jnp.float32),
        pltpu.SemaphoreType.DMA,
        pltpu.SemaphoreType.DMA,
    ],
    compiler_params=_SC_PARAMS,
)
def _sc_degree(edge_hbm, out_hbm, di_v, ones_v, zeros_v, acc_sh, isem, dsem):
    cid = lax.axis_index("c")
    sid = lax.axis_index("s")
    wid = sid * NC + cid

    _stage_dst(edge_hbm, di_v, wid * EPW, isem)

    @pl.loop(0, G)
    def _(i):
        ones_v[i] = jnp.ones((L,), jnp.float32)

    @pl.loop(0, ZR)
    def _(i):
        zeros_v[i] = jnp.zeros((L,), jnp.float32)

    @pl.loop(0, RPT, step=ZR)
    def _(r):
        pltpu.sync_copy(zeros_v, acc_sh.at[pl.ds(sid * RPT + r, ZR)])

    plsc.subcore_barrier()

    @pl.loop(0, NG, step=NB)
    def _(base):
        for b in range(NB):
            pltpu.async_copy(ones_v, acc_sh.at[di_v.at[base + b]], dsem,
                             add=True)
        for b in range(NB):
            pltpu.make_async_copy(ones_v, acc_sh.at[di_v.at[base + b]],
                                  dsem).wait()

    plsc.subcore_barrier()
    pltpu.sync_copy(acc_sh.at[pl.ds(sid * RPT, RPT)],
                    out_hbm.at[cid, pl.ds(sid * RPT, RPT)])


@functools.partial(
    pl.kernel,
    mesh=_MESH,
    out_type=jax.ShapeDtypeStruct((NC, NPAD, HID), jnp.float32),
    scratch_types=[
        pltpu.VMEM((NG * G,), jnp.int32),
        pltpu.VMEM((NG, G), jnp.int32),
        pltpu.VMEM((2 * NB, G, HID), jnp.float32),
        pltpu.VMEM((ZR, HID), jnp.float32),
        pltpu.VMEM_SHARED((NPAD, HID), jnp.float32),
        pltpu.VMEM_SHARED((N, HID), jnp.float32),
        pltpu.SemaphoreType.DMA,
        pltpu.SemaphoreType.DMA,
        pltpu.SemaphoreType.DMA,
        pltpu.SemaphoreType.DMA((2 * NB,)),
        pltpu.SemaphoreType.DMA((2 * NB,)),
    ],
    compiler_params=_SC_PARAMS,
)
def _sc_aggregate(table_hbm, edge_hbm, out_hbm,
                  si_v, di_v, rows_v, zeros_v, acc_sh, tbl_sh,
                  isem_s, isem_d, tsem, gsem, ssem):
    cid = lax.axis_index("c")
    sid = lax.axis_index("s")
    wid = sid * NC + cid
    ebase = wid * EPW

    cp_s = pltpu.async_copy(edge_hbm.at[0, pl.ds(ebase, EPW)],
                            si_v.at[pl.ds(0, EPW)], isem_s)
    cp_t = pltpu.async_copy(table_hbm.at[pl.ds(sid * TSL, TSL)],
                            tbl_sh.at[pl.ds(sid * TSL, TSL)], tsem)

    _stage_dst(edge_hbm, di_v, ebase, isem_d)

    for k in range((NG * G - EPW) // L):
        si_v[pl.ds(EPW + k * L, L)] = jnp.zeros((L,), jnp.int32)

    @pl.loop(0, ZR)
    def _(i):
        zeros_v[i] = jnp.zeros((L,), jnp.float32)

    @pl.loop(0, RPT, step=ZR)
    def _(r):
        pltpu.sync_copy(zeros_v, acc_sh.at[pl.ds(sid * RPT + r, ZR)])

    cp_s.wait()
    cp_t.wait()
    plsc.subcore_barrier()

    def _gather(g, buf):
        pltpu.async_copy(tbl_sh.at[si_v.at[pl.ds(g * G, G)]],
                         rows_v.at[buf], gsem.at[buf])

    def _wait_gather(g, buf):
        pltpu.make_async_copy(tbl_sh.at[si_v.at[pl.ds(g * G, G)]],
                              rows_v.at[buf], gsem.at[buf]).wait()

    def _scatter(g, buf):
        pltpu.async_copy(rows_v.at[buf], acc_sh.at[di_v.at[g]],
                         ssem.at[buf], add=True)

    def _wait_scatter(g, buf):
        pltpu.make_async_copy(rows_v.at[buf], acc_sh.at[di_v.at[g]],
                              ssem.at[buf]).wait()

    for b in range(NB):
        _gather(b, b)

    for b in range(NB):
        _wait_gather(b, b)
        _scatter(b, b)
        _gather(NB + b, NB + b)

    @pl.loop(0, (PHASES - 2) // 2)
    def _(p):
        for half, cur0 in ((0, NB), (1, 0)):
            base = (2 * p + 1 + half) * NB
            nxt0 = NB - cur0
            for b in range(NB):
                _wait_scatter(base - NB + b, nxt0 + b)
                _wait_gather(base + b, cur0 + b)
                _scatter(base + b, cur0 + b)
                _gather(base + NB + b, nxt0 + b)

    base = (PHASES - 1) * NB
    cur0 = ((PHASES - 1) % 2) * NB
    nxt0 = NB - cur0
    for b in range(NB):
        _wait_scatter(base - NB + b, nxt0 + b)
        _wait_gather(base + b, cur0 + b)
        _scatter(base + b, cur0 + b)
    for b in range(NB):
        _wait_scatter(base + b, cur0 + b)

    plsc.subcore_barrier()
    pltpu.sync_copy(acc_sh.at[pl.ds(sid * RPT, RPT)],
                    out_hbm.at[cid, pl.ds(sid * RPT, RPT)])


def _tc_mm1_body(xw_ref, w1b_ref, hw_ref):
    hw_ref[...] = jnp.dot(xw_ref[...], w1b_ref[...],
                          preferred_element_type=jnp.float32)


def _tc_scale_body(hw_ref, degw_ref, hsw_ref, dinw_ref):
    deg = degw_ref[0, :NWIDE, :] + degw_ref[1, :NWIDE, :] + 1.0
    dinv = lax.rsqrt(deg)
    hsw_ref[...] = hw_ref[...] * dinv
    dinw_ref[...] = dinv


def _tc2_body(aggw_ref, hsw_ref, dinw_ref, w2b_ref, b1w_ref, hs2w_ref):
    a = aggw_ref[0, :NWIDE, :] + aggw_ref[1, :NWIDE, :] + hsw_ref[...]
    h1 = jnp.maximum(dinw_ref[...] * a + b1w_ref[...], 0.0)
    hs2w_ref[...] = jnp.dot(h1, w2b_ref[...],
                            preferred_element_type=jnp.float32) * dinw_ref[...]


def _tc3_body(aggw_ref, hsw_ref, dinw_ref, b2w_ref, m_ref, outw_ref):
    a = aggw_ref[0, :NWIDE, :] + aggw_ref[1, :NWIDE, :] + hsw_ref[...]
    pre = dinw_ref[...] * a + b2w_ref[...]
    m = jnp.max(pre, axis=1, keepdims=True)
    sh = pre - m
    sw = jnp.dot(jnp.exp(sh), m_ref[...], preferred_element_type=jnp.float32)
    outw_ref[...] = sh - jnp.log(sw)


def kernel(x, edge_index, W1, b1, W2, b2):
    eye8 = jnp.eye(8, dtype=jnp.float32)
    w1b = jnp.kron(eye8, W1)
    w2b = jnp.kron(eye8, W2)
    mseg = jnp.kron(eye8, jnp.ones((NCLS, NCLS), jnp.float32))
    b1w = jnp.tile(b1, 8).reshape(1, 8 * HID)
    b2w = jnp.tile(b2, 8).reshape(1, 8 * NCLS)
    xw = x.reshape(NWIDE, KWIDE)

    deg2 = _sc_degree(edge_index)

    hw = pl.pallas_call(
        _tc_mm1_body,
        out_shape=jax.ShapeDtypeStruct((NWIDE, 8 * HID), jnp.float32),
    )(xw, w1b)

    hs1w, dinw = pl.pallas_call(
        _tc_scale_body,
        out_shape=[jax.ShapeDtypeStruct((NWIDE, 8 * HID), jnp.float32),
                   jax.ShapeDtypeStruct((NWIDE, 8 * HID), jnp.float32)],
    )(hw, deg2.reshape(NC, DPADW, 8 * HID))

    agg1 = _sc_aggregate(hs1w.reshape(N, HID), edge_index)

    hs2w = pl.pallas_call(
        _tc2_body,
        out_shape=jax.ShapeDtypeStruct((NWIDE, 8 * HID), jnp.float32),
    )(agg1.reshape(NC, DPADW, 8 * HID), hs1w, dinw, w2b, b1w)

    agg2 = _sc_aggregate(hs2w.reshape(N, HID), edge_index)

    outw = pl.pallas_call(
        _tc3_body,
        out_shape=jax.ShapeDtypeStruct((NWIDE, 8 * NCLS), jnp.float32),
    )(agg2.reshape(NC, DPADW, 8 * NCLS), hs2w, dinw, b2w, mseg)

    return outw.reshape(N, NCLS)


# speedup vs baseline: 1.2317x; 1.0013x over previous
import functools

import jax
import jax.numpy as jnp
from jax import lax
from jax.experimental import pallas as pl
from jax.experimental.pallas import tpu as pltpu
from jax.experimental.pallas import tpu_sc as plsc

N = 10000
E = 320000
F_IN = 128
HID = 16
NCLS = 16

NC = 2
NS = 16
L = 16
NW = NC * NS
EPW = E // NW
G = 128
NB = 4
DNB = 8
NG = 80
PHASES = NG // NB
FULL_ROWS = EPW // G
TAIL = EPW - FULL_ROWS * G
NPAD = 10240
RPT = NPAD // NS
TSL = N // NS
ZR = 64

NWIDE = N // 8
KWIDE = F_IN * 8
DPADW = NPAD // 8

_MESH = plsc.VectorSubcoreMesh(core_axis_name="c", subcore_axis_name="s")
_SC_PARAMS = pltpu.CompilerParams(use_tc_tiling_on_sc=False)


def _stage_dst(edge_hbm, di_v, ebase, isem):
    @pl.loop(0, FULL_ROWS)
    def _(g):
        pltpu.async_copy(edge_hbm.at[1, pl.ds(ebase + g * G, G)],
                         di_v.at[g], isem)

    pltpu.sync_copy(edge_hbm.at[1, pl.ds(ebase + FULL_ROWS * G, TAIL)],
                    di_v.at[FULL_ROWS, pl.ds(0, TAIL)])
    for k in range(TAIL // L, G // L):
        di_v[FULL_ROWS, pl.ds(k * L, L)] = jnp.full((L,), N, jnp.int32)
    for g in range(FULL_ROWS + 1, NG):
        for k in range(G // L):
            di_v[g, pl.ds(k * L, L)] = jnp.full((L,), N, jnp.int32)

    @pl.loop(0, FULL_ROWS)
    def _(g):
        pltpu.make_async_copy(edge_hbm.at[1, pl.ds(ebase + g * G, G)],
                              di_v.at[g], isem).wait()


@functools.partial(
    pl.kernel,
    mesh=_MESH,
    out_type=jax.ShapeDtypeStruct((NC, NPAD, HID), jnp.float32),
    scratch_types=[
        pltpu.VMEM((NG, G), jnp.int32),
        pltpu.VMEM((G, HID), jnp.float32),
        pltpu.VMEM((ZR, HID), jnp.float32),
        pltpu.VMEM_SHARED((NPAD, HID), jnp.float32),
        pltpu.SemaphoreType.DMA,
        pltpu.SemaphoreType.DMA,
    ],
    compiler_params=_SC_PARAMS,
)
def _sc_degree(edge_hbm, out_hbm, di_v, ones_v, zeros_v, acc_sh, isem, dsem):
    cid = lax.axis_index("c")
    sid = lax.axis_index("s")
    wid = sid * NC + cid

    _stage_dst(edge_hbm, di_v, wid * EPW, isem)

    @pl.loop(0, G)
    def _(i):
        ones_v[i] = jnp.ones((L,), jnp.float32)

    @pl.loop(0, ZR)
    def _(i):
        zeros_v[i] = jnp.zeros((L,), jnp.float32)

    @pl.loop(0, RPT, step=ZR)
    def _(r):
        pltpu.sync_copy(zeros_v, acc_sh.at[pl.ds(sid * RPT + r, ZR)])

    plsc.subcore_barrier()

    @pl.loop(0, NG, step=DNB)
    def _(base):
        for b in range(DNB):
            pltpu.async_copy(ones_v, acc_sh.at[di_v.at[base + b]], dsem,
                             add=True)
        for b in range(DNB):
            pltpu.make_async_copy(ones_v, acc_sh.at[di_v.at[base + b]],
                                  dsem).wait()

    plsc.subcore_barrier()
    pltpu.sync_copy(acc_sh.at[pl.ds(sid * RPT, RPT)],
                    out_hbm.at[cid, pl.ds(sid * RPT, RPT)])


@functools.partial(
    pl.kernel,
    mesh=_MESH,
    out_type=jax.ShapeDtypeStruct((NC, NPAD, HID), jnp.float32),
    scratch_types=[
        pltpu.VMEM((NG * G,), jnp.int32),
        pltpu.VMEM((NG, G), jnp.int32),
        pltpu.VMEM((2 * NB, G, HID), jnp.float32),
        pltpu.VMEM((ZR, HID), jnp.float32),
        pltpu.VMEM_SHARED((NPAD, HID), jnp.float32),
        pltpu.VMEM_SHARED((N, HID), jnp.float32),
        pltpu.SemaphoreType.DMA,
        pltpu.SemaphoreType.DMA,
        pltpu.SemaphoreType.DMA,
        pltpu.SemaphoreType.DMA((2 * NB,)),
        pltpu.SemaphoreType.DMA((2 * NB,)),
    ],
    compiler_params=_SC_PARAMS,
)
def _sc_aggregate(table_hbm, edge_hbm, out_hbm,
                  si_v, di_v, rows_v, zeros_v, acc_sh, tbl_sh,
                  isem_s, isem_d, tsem, gsem, ssem):
    cid = lax.axis_index("c")
    sid = lax.axis_index("s")
    wid = sid * NC + cid
    ebase = wid * EPW

    cp_s = pltpu.async_copy(edge_hbm.at[0, pl.ds(ebase, EPW)],
                            si_v.at[pl.ds(0, EPW)], isem_s)
    cp_t = pltpu.async_copy(table_hbm.at[pl.ds(sid * TSL, TSL)],
                            tbl_sh.at[pl.ds(sid * TSL, TSL)], tsem)

    _stage_dst(edge_hbm, di_v, ebase, isem_d)

    for k in range((NG * G - EPW) // L):
        si_v[pl.ds(EPW + k * L, L)] = jnp.zeros((L,), jnp.int32)

    @pl.loop(0, ZR)
    def _(i):
        zeros_v[i] = jnp.zeros((L,), jnp.float32)

    @pl.loop(0, RPT, step=ZR)
    def _(r):
        pltpu.sync_copy(zeros_v, acc_sh.at[pl.ds(sid * RPT + r, ZR)])

    cp_s.wait()
    cp_t.wait()
    plsc.subcore_barrier()

    def _gather(g, buf):
        pltpu.async_copy(tbl_sh.at[si_v.at[pl.ds(g * G, G)]],
                         rows_v.at[buf], gsem.at[buf])

    def _wait_gather(g, buf):
        pltpu.make_async_copy(tbl_sh.at[si_v.at[pl.ds(g * G, G)]],
                              rows_v.at[buf], gsem.at[buf]).wait()

    def _scatter(g, buf):
        pltpu.async_copy(rows_v.at[buf], acc_sh.at[di_v.at[g]],
                         ssem.at[buf], add=True)

    def _wait_scatter(g, buf):
        pltpu.make_async_copy(rows_v.at[buf], acc_sh.at[di_v.at[g]],
                              ssem.at[buf]).wait()

    for b in range(NB):
        _gather(b, b)

    for b in range(NB):
        _wait_gather(b, b)
        _scatter(b, b)
        _gather(NB + b, NB + b)

    @pl.loop(0, (PHASES - 2) // 2)
    def _(p):
        for half, cur0 in ((0, NB), (1, 0)):
            base = (2 * p + 1 + half) * NB
            nxt0 = NB - cur0
            for b in range(NB):
                _wait_scatter(base - NB + b, nxt0 + b)
                _wait_gather(base + b, cur0 + b)
                _scatter(base + b, cur0 + b)
                _gather(base + NB + b, nxt0 + b)

    base = (PHASES - 1) * NB
    cur0 = ((PHASES - 1) % 2) * NB
    nxt0 = NB - cur0
    for b in range(NB):
        _wait_scatter(base - NB + b, nxt0 + b)
        _wait_gather(base + b, cur0 + b)
        _scatter(base + b, cur0 + b)
    for b in range(NB):
        _wait_scatter(base + b, cur0 + b)

    plsc.subcore_barrier()
    pltpu.sync_copy(acc_sh.at[pl.ds(sid * RPT, RPT)],
                    out_hbm.at[cid, pl.ds(sid * RPT, RPT)])


def _tc_mm1_body(xw_ref, w1b_ref, hw_ref):
    hw_ref[...] = jnp.dot(xw_ref[...], w1b_ref[...],
                          preferred_element_type=jnp.float32)


def _tc_scale_body(hw_ref, degw_ref, hsw_ref, dinw_ref):
    deg = degw_ref[0, :NWIDE, :] + degw_ref[1, :NWIDE, :] + 1.0
    dinv = lax.rsqrt(deg)
    hsw_ref[...] = hw_ref[...] * dinv
    dinw_ref[...] = dinv


def _tc2_body(aggw_ref, hsw_ref, dinw_ref, w2b_ref, b1w_ref, hs2w_ref):
    a = aggw_ref[0, :NWIDE, :] + aggw_ref[1, :NWIDE, :] + hsw_ref[...]
    h1 = jnp.maximum(dinw_ref[...] * a + b1w_ref[...], 0.0)
    hs2w_ref[...] = jnp.dot(h1, w2b_ref[...],
                            preferred_element_type=jnp.float32) * dinw_ref[...]


def _tc3_body(aggw_ref, hsw_ref, dinw_ref, b2w_ref, m_ref, outw_ref):
    a = aggw_ref[0, :NWIDE, :] + aggw_ref[1, :NWIDE, :] + hsw_ref[...]
    pre = dinw_ref[...] * a + b2w_ref[...]
    m = jnp.max(pre, axis=1, keepdims=True)
    sh = pre - m
    sw = jnp.dot(jnp.exp(sh), m_ref[...], preferred_element_type=jnp.float32)
    outw_ref[...] = sh - jnp.log(sw)


def kernel(x, edge_index, W1, b1, W2, b2):
    eye8 = jnp.eye(8, dtype=jnp.float32)
    w1b = jnp.kron(eye8, W1)
    w2b = jnp.kron(eye8, W2)
    mseg = jnp.kron(eye8, jnp.ones((NCLS, NCLS), jnp.float32))
    b1w = jnp.tile(b1, 8).reshape(1, 8 * HID)
    b2w = jnp.tile(b2, 8).reshape(1, 8 * NCLS)
    xw = x.reshape(NWIDE, KWIDE)

    deg2 = _sc_degree(edge_index)

    hw = pl.pallas_call(
        _tc_mm1_body,
        out_shape=jax.ShapeDtypeStruct((NWIDE, 8 * HID), jnp.float32),
    )(xw, w1b)

    hs1w, dinw = pl.pallas_call(
        _tc_scale_body,
        out_shape=[jax.ShapeDtypeStruct((NWIDE, 8 * HID), jnp.float32),
                   jax.ShapeDtypeStruct((NWIDE, 8 * HID), jnp.float32)],
    )(hw, deg2.reshape(NC, DPADW, 8 * HID))

    agg1 = _sc_aggregate(hs1w.reshape(N, HID), edge_index)

    hs2w = pl.pallas_call(
        _tc2_body,
        out_shape=jax.ShapeDtypeStruct((NWIDE, 8 * HID), jnp.float32),
    )(agg1.reshape(NC, DPADW, 8 * HID), hs1w, dinw, w2b, b1w)

    agg2 = _sc_aggregate(hs2w.reshape(N, HID), edge_index)

    outw = pl.pallas_call(
        _tc3_body,
        out_shape=jax.ShapeDtypeStruct((NWIDE, 8 * NCLS), jnp.float32),
    )(agg2.reshape(NC, DPADW, 8 * NCLS), hs2w, dinw, b2w, mseg)

    return outw.reshape(N, NCLS)
